# Initial kernel scaffold; baseline (speedup 1.0000x reference)
import functools

import jax
import jax.numpy as jnp
from jax import lax
from jax.experimental import pallas as pl
from jax.experimental.pallas import tpu as pltpu
from jax.experimental.pallas import tpu_sc as plsc


def _grid_params(total: int):
    info = plsc.get_sparse_core_info()
    nc, ns = info.num_cores, info.num_subcores
    nw = nc * ns
    assert total % nw == 0
    bpw = total // nw
    nchunk = 8
    while bpw % nchunk:
        nchunk += 1
    csz = bpw // nchunk
    assert csz % 8 == 0
    return nc, ns, nw, bpw, nchunk, csz


@functools.lru_cache(maxsize=None)
def _build_gather(total: int, dim: int):
    nc, ns, nw, bpw, nchunk, csz = _grid_params(total)
    mesh = plsc.VectorSubcoreMesh(core_axis_name="core", subcore_axis_name="subcore")

    @functools.partial(
        pl.kernel,
        mesh=mesh,
        out_type=jax.ShapeDtypeStruct((total, dim), jnp.float32),
        scratch_types=[
            pltpu.VMEM((nchunk, csz), jnp.int32),
            pltpu.VMEM((csz, dim), jnp.float32),
            pltpu.SemaphoreType.DMA,
        ],
    )
    def gather(table_hbm, idx_hbm, out_hbm, idx_v, rows_v, sem):
        wid = lax.axis_index("subcore") * nc + lax.axis_index("core")
        base = wid * bpw
        pltpu.sync_copy(idx_hbm.at[wid], idx_v)
        for j in range(nchunk):
            pltpu.async_copy(table_hbm.at[idx_v.at[j]], rows_v, sem).wait()
            pltpu.sync_copy(rows_v, out_hbm.at[pl.ds(base + j * csz, csz)])

    return gather


def kernel(indices, tables):
    b, s = indices.shape
    factor, split, dim = tables.shape
    total = b * s
    flat_table = tables.reshape(factor * split, dim)
    _, _, nw, _, nchunk, csz = _grid_params(total)
    idx = indices.reshape(nw, nchunk, csz).astype(jnp.int32)
    out = _build_gather(total, dim)(flat_table, idx)
    return out.reshape(b, s, dim)


# speedup vs baseline: 71.3719x; 71.3719x over previous
import functools

import jax
import jax.numpy as jnp
from jax import lax
from jax.experimental import pallas as pl
from jax.experimental.pallas import tpu as pltpu
from jax.experimental.pallas import tpu_sc as plsc


def _grid_params(total: int):
    info = plsc.get_sparse_core_info()
    nc, ns = info.num_cores, info.num_subcores
    nw = nc * ns
    assert total % nw == 0
    bpw = total // nw
    nchunk = 8
    while bpw % nchunk:
        nchunk += 1
    csz = bpw // nchunk
    assert csz % 8 == 0
    return nc, ns, nw, bpw, nchunk, csz


@functools.lru_cache(maxsize=None)
def _build_gather(total: int, dim: int):
    nc, ns, nw, bpw, nchunk, csz = _grid_params(total)
    mesh = plsc.VectorSubcoreMesh(core_axis_name="core", subcore_axis_name="subcore")

    @functools.partial(
        pl.kernel,
        mesh=mesh,
        out_type=jax.ShapeDtypeStruct((total, dim), jnp.float32),
        compiler_params=pltpu.CompilerParams(use_tc_tiling_on_sc=False),
        scratch_types=[
            pltpu.VMEM((nchunk, csz), jnp.int32),
            pltpu.VMEM((csz, dim), jnp.float32),
            pltpu.SemaphoreType.DMA,
        ],
    )
    def gather(table_hbm, idx_hbm, out_hbm, idx_v, rows_v, sem):
        wid = lax.axis_index("subcore") * nc + lax.axis_index("core")
        base = wid * bpw
        pltpu.sync_copy(idx_hbm.at[wid], idx_v)
        for j in range(nchunk):
            pltpu.async_copy(table_hbm.at[idx_v.at[j]], rows_v, sem).wait()
            pltpu.sync_copy(rows_v, out_hbm.at[pl.ds(base + j * csz, csz)])

    return gather


def kernel(indices, tables):
    b, s = indices.shape
    factor, split, dim = tables.shape
    total = b * s
    flat_table = tables.reshape(factor * split, dim)
    _, _, nw, _, nchunk, csz = _grid_params(total)
    idx = indices.reshape(nw, nchunk, csz).astype(jnp.int32)
    out = _build_gather(total, dim)(flat_table, idx)
    return out.reshape(b, s, dim)


# speedup vs baseline: 71.7860x; 1.0058x over previous
import functools

import jax
import jax.numpy as jnp
from jax import lax
from jax.experimental import pallas as pl
from jax.experimental.pallas import tpu as pltpu
from jax.experimental.pallas import tpu_sc as plsc


def _grid_params(total: int):
    info = plsc.get_sparse_core_info()
    nc, ns = info.num_cores, info.num_subcores
    nw = nc * ns
    assert total % nw == 0
    bpw = total // nw
    nchunk = 8
    while bpw % nchunk:
        nchunk += 1
    csz = bpw // nchunk
    assert csz % 8 == 0
    return nc, ns, nw, bpw, nchunk, csz


@functools.lru_cache(maxsize=None)
def _build_gather(total: int, dim: int):
    nc, ns, nw, bpw, nchunk, csz = _grid_params(total)
    mesh = plsc.VectorSubcoreMesh(core_axis_name="core", subcore_axis_name="subcore")

    @functools.partial(
        pl.kernel,
        mesh=mesh,
        out_type=jax.ShapeDtypeStruct((total, dim), jnp.float32),
        compiler_params=pltpu.CompilerParams(use_tc_tiling_on_sc=False),
        scratch_types=[
            pltpu.VMEM((nchunk, csz), jnp.int32),
            pltpu.VMEM((csz, dim), jnp.float32),
            pltpu.VMEM((csz, dim), jnp.float32),
            pltpu.SemaphoreType.DMA,
            pltpu.SemaphoreType.DMA,
            pltpu.SemaphoreType.DMA,
            pltpu.SemaphoreType.DMA,
        ],
    )
    def gather(table_hbm, idx_hbm, out_hbm, idx_v, rows0, rows1, gs0, gs1, ws0, ws1):
        bufs = (rows0, rows1)
        gsems = (gs0, gs1)
        wsems = (ws0, ws1)
        wid = lax.axis_index("subcore") * nc + lax.axis_index("core")
        base = wid * bpw
        pltpu.sync_copy(idx_hbm.at[wid], idx_v)

        def gstart(j):
            return pltpu.async_copy(
                table_hbm.at[idx_v.at[j]], bufs[j % 2], gsems[j % 2]
            )

        def wstart(j):
            return pltpu.async_copy(
                bufs[j % 2], out_hbm.at[pl.ds(base + j * csz, csz)], wsems[j % 2]
            )

        gh = [None] * nchunk
        wh = [None] * nchunk
        gh[0] = gstart(0)
        if nchunk > 1:
            gh[1] = gstart(1)
        for j in range(nchunk):
            gh[j].wait()
            wh[j] = wstart(j)
            if j + 2 < nchunk:
                wh[j].wait()
                gh[j + 2] = gstart(j + 2)
        for j in range(max(0, nchunk - 2), nchunk):
            wh[j].wait()

    return gather


def kernel(indices, tables):
    b, s = indices.shape
    factor, split, dim = tables.shape
    total = b * s
    flat_table = tables.reshape(factor * split, dim)
    _, _, nw, _, nchunk, csz = _grid_params(total)
    idx = indices.reshape(nw, nchunk, csz).astype(jnp.int32)
    out = _build_gather(total, dim)(flat_table, idx)
    return out.reshape(b, s, dim)
